# baseline (speedup 1.0000x reference)
import functools

import jax
import jax.numpy as jnp
from jax import lax
from jax.experimental import pallas as pl
from jax.experimental.pallas import tpu as pltpu
from jax.experimental.pallas import tpu_sc as plsc

_B, _T, _K = 32, 4096, 256
_L = 16
_TC = 128
_CHUNK_W = _TC * _K
_NCHUNK = _T // _TC
_NBUF = 2


def _positions(idx_v, g, k):
    io = lax.iota(jnp.int32, _L)
    tok = g * _TC + k * _L + io
    src = jnp.minimum(tok + 1, _T - 1)
    s = plsc.load_gather(idx_v, [src])
    s = s + jnp.where(tok + 1 > _T - 1, 1, 0)
    s = jnp.where(s > _K - 1, 0, s)
    return (k * _L + io) * _K + s


def _sc_body(inp_hbm, out_hbm, idx_v, buf0, buf1, sem0, sem1):
    wid = lax.axis_index("s") * 2 + lax.axis_index("c")
    bufs = (buf0, buf1)
    sems = (sem0, sem1)
    ones = jnp.full((_L,), 1.0, jnp.float32)
    zeros = jnp.zeros((_L,), jnp.float32)

    pltpu.sync_copy(inp_hbm.at[pl.ds(wid * _T, _T)], idx_v)

    def _zero(i, _):
        buf0[pl.ds(i * _L, _L)] = zeros
        buf1[pl.ds(i * _L, _L)] = zeros
        return 0
    lax.fori_loop(0, _CHUNK_W // _L, _zero, 0)

    out_base = wid * (_T * _K)

    def _fire(g, b):
        for k in range(_TC // _L):
            plsc.store_scatter(bufs[b], [_positions(idx_v, g, k)], ones)
        return pltpu.make_async_copy(
            bufs[b],
            out_hbm.at[pl.ds(out_base + g * _CHUNK_W, _CHUNK_W)],
            sems[b],
        )

    for b in range(_NBUF):
        _fire(jnp.int32(b), b).start()

    def _step(q, _):
        for b in range(_NBUF):
            g = q * _NBUF + b
            pltpu.make_async_copy(
                bufs[b],
                out_hbm.at[pl.ds(out_base + (g - _NBUF) * _CHUNK_W, _CHUNK_W)],
                sems[b],
            ).wait()
            for k in range(_TC // _L):
                plsc.store_scatter(bufs[b], [_positions(idx_v, g - _NBUF, k)], zeros)
            _fire(g, b).start()
        return 0

    lax.fori_loop(1, _NCHUNK // _NBUF, _step, 0)

    for b in range(_NBUF):
        g = _NCHUNK - _NBUF + b
        pltpu.make_async_copy(
            bufs[b],
            out_hbm.at[pl.ds(out_base + g * _CHUNK_W, _CHUNK_W)],
            sems[b],
        ).wait()


def kernel(inputs):
    B, T = inputs.shape
    flat = inputs.astype(jnp.int32).reshape(B * T)
    run = pl.kernel(
        _sc_body,
        mesh=plsc.VectorSubcoreMesh(core_axis_name="c", subcore_axis_name="s"),
        compiler_params=pltpu.CompilerParams(needs_layout_passes=False),
        out_type=jax.ShapeDtypeStruct((_B * _T * _K,), jnp.float32),
        scratch_types=[
            pltpu.VMEM((_T,), jnp.int32),
            pltpu.VMEM((_CHUNK_W,), jnp.float32),
            pltpu.VMEM((_CHUNK_W,), jnp.float32),
            pltpu.SemaphoreType.DMA,
            pltpu.SemaphoreType.DMA,
        ],
    )
    return run(flat).reshape(B, T, _K)


# speedup vs baseline: 1.0272x; 1.0272x over previous
import functools

import jax
import jax.numpy as jnp
from jax import lax
from jax.experimental import pallas as pl
from jax.experimental.pallas import tpu as pltpu
from jax.experimental.pallas import tpu_sc as plsc

_B, _T, _K = 32, 4096, 256
_L = 16
_TC = 64
_CHUNK_W = _TC * _K
_NCHUNK = _T // _TC
_NBUF = 4


def _positions(idx_v, g, k):
    io = lax.iota(jnp.int32, _L)
    tok = g * _TC + k * _L + io
    src = jnp.minimum(tok + 1, _T - 1)
    s = plsc.load_gather(idx_v, [src])
    s = s + jnp.where(tok + 1 > _T - 1, 1, 0)
    s = jnp.where(s > _K - 1, 0, s)
    return (k * _L + io) * _K + s


def _sc_body(inp_hbm, out_hbm, idx_v, buf0, buf1, buf2, buf3,
             sem0, sem1, sem2, sem3):
    wid = lax.axis_index("s") * 2 + lax.axis_index("c")
    bufs = (buf0, buf1, buf2, buf3)
    sems = (sem0, sem1, sem2, sem3)
    ones = jnp.full((_L,), 1.0, jnp.float32)
    zeros = jnp.zeros((_L,), jnp.float32)

    pltpu.sync_copy(inp_hbm.at[pl.ds(wid * _T, _T)], idx_v)

    def _zero(i, _):
        buf0[pl.ds(i * _L, _L)] = zeros
        buf1[pl.ds(i * _L, _L)] = zeros
        buf2[pl.ds(i * _L, _L)] = zeros
        buf3[pl.ds(i * _L, _L)] = zeros
        return 0
    lax.fori_loop(0, _CHUNK_W // _L, _zero, 0)

    out_base = wid * (_T * _K)

    def _fire(g, b):
        for k in range(_TC // _L):
            plsc.store_scatter(bufs[b], [_positions(idx_v, g, k)], ones)
        return pltpu.make_async_copy(
            bufs[b],
            out_hbm.at[pl.ds(out_base + g * _CHUNK_W, _CHUNK_W)],
            sems[b],
        )

    for b in range(_NBUF):
        _fire(jnp.int32(b), b).start()

    def _step(q, _):
        for b in range(_NBUF):
            g = q * _NBUF + b
            pltpu.make_async_copy(
                bufs[b],
                out_hbm.at[pl.ds(out_base + (g - _NBUF) * _CHUNK_W, _CHUNK_W)],
                sems[b],
            ).wait()
            for k in range(_TC // _L):
                plsc.store_scatter(bufs[b], [_positions(idx_v, g - _NBUF, k)], zeros)
            _fire(g, b).start()
        return 0

    lax.fori_loop(1, _NCHUNK // _NBUF, _step, 0)

    for b in range(_NBUF):
        g = _NCHUNK - _NBUF + b
        pltpu.make_async_copy(
            bufs[b],
            out_hbm.at[pl.ds(out_base + g * _CHUNK_W, _CHUNK_W)],
            sems[b],
        ).wait()


def kernel(inputs):
    B, T = inputs.shape
    flat = inputs.astype(jnp.int32).reshape(B * T)
    run = pl.kernel(
        _sc_body,
        mesh=plsc.VectorSubcoreMesh(core_axis_name="c", subcore_axis_name="s"),
        compiler_params=pltpu.CompilerParams(needs_layout_passes=False),
        out_type=jax.ShapeDtypeStruct((_B * _T * _K,), jnp.float32),
        scratch_types=[
            pltpu.VMEM((_T,), jnp.int32),
            pltpu.VMEM((_CHUNK_W,), jnp.float32),
            pltpu.VMEM((_CHUNK_W,), jnp.float32),
            pltpu.VMEM((_CHUNK_W,), jnp.float32),
            pltpu.VMEM((_CHUNK_W,), jnp.float32),
            pltpu.SemaphoreType.DMA,
            pltpu.SemaphoreType.DMA,
            pltpu.SemaphoreType.DMA,
            pltpu.SemaphoreType.DMA,
        ],
    )
    return run(flat).reshape(B, T, _K)


# speedup vs baseline: 4.8566x; 4.7280x over previous
import functools

import jax
import jax.numpy as jnp
from jax.experimental import pallas as pl
from jax.experimental.pallas import tpu as pltpu


def _onehot_body(rows_ref, out_ref, raw_ref, *, n_blk, n_classes, n_batch):
    j = pl.program_id(0)
    h = pl.program_id(1)
    n_j = pl.num_programs(0)
    half = n_batch // 2

    @pl.when((j == 0) & (h == 0))
    def _prep_first():
        raw_ref[pl.ds(0, 2 * n_blk), :] = jnp.transpose(
            rows_ref[:, pl.ds(0, 2 * n_blk)], (1, 0))

    @pl.when((h == 1) & (j < n_j - 2))
    def _prep_next():
        p = j + 2
        raw_ref[pl.ds(p * n_blk, n_blk), :] = jnp.transpose(
            rows_ref[:, pl.ds(p * n_blk, n_blk)], (1, 0))

    blk = raw_ref[pl.ds(j * n_blk, n_blk), :]
    nxt = jax.lax.rem(j + 1, n_j)
    nxt_row = raw_ref[pl.ds(nxt * n_blk, 8), :][0:1]
    nxt_val = jnp.where(j == n_j - 1, blk[-1:, :] + 1, nxt_row)
    shifted = jnp.concatenate([blk[1:, :], nxt_val], axis=0)
    shifted = jnp.where(shifted > n_classes - 1, 0, shifted)
    iota = jax.lax.broadcasted_iota(jnp.int32, (n_blk, n_classes), 1)
    for hh in range(2):
        @pl.when(h == hh)
        def _emit(hh=hh):
            for b in range(half):
                col = shifted[:, hh * half + b:hh * half + b + 1]
                out_ref[b] = jnp.where(col == iota, jnp.float32(1.0),
                                       jnp.float32(0.0))


def kernel(inputs):
    B, T = inputs.shape
    K = 256
    N = 512
    C = T // N
    return pl.pallas_call(
        functools.partial(_onehot_body, n_blk=N, n_classes=K, n_batch=B),
        grid=(C, 2),
        in_specs=[pl.BlockSpec((B, T), lambda j, h: (0, 0))],
        out_specs=pl.BlockSpec((B // 2, N, K), lambda j, h: (h, j, 0)),
        out_shape=jax.ShapeDtypeStruct((B, T, K), jnp.float32),
        scratch_shapes=[pltpu.VMEM((T, B), jnp.int32)],
        compiler_params=pltpu.CompilerParams(
            dimension_semantics=("arbitrary", "arbitrary"),
        ),
    )(inputs.astype(jnp.int32))


# speedup vs baseline: 4.9023x; 1.0094x over previous
import functools

import jax
import jax.numpy as jnp
from jax.experimental import pallas as pl
from jax.experimental.pallas import tpu as pltpu


def _onehot_body(rows_ref, out_ref, raw_ref, *, n_blk, n_classes, n_batch,
                 n_pieces, piece):
    j = pl.program_id(0)
    n_j = pl.num_programs(0)

    for p in range(n_pieces):
        @pl.when(j == p)
        def _prep(p=p):
            raw_ref[pl.ds(p * piece, piece), :] = jnp.transpose(
                rows_ref[:, pl.ds(p * piece, piece)], (1, 0))

    blk = raw_ref[pl.ds(j * n_blk, n_blk), :]
    nxt = jax.lax.rem(j + 1, n_j)
    nxt_row = raw_ref[pl.ds(nxt * n_blk, 8), :][0:1]
    nxt_val = jnp.where(j == n_j - 1, blk[-1:, :] + 1, nxt_row)
    shifted = jnp.concatenate([blk[1:, :], nxt_val], axis=0)
    shifted = jnp.where(shifted > n_classes - 1, 0, shifted)
    iota = jax.lax.broadcasted_iota(jnp.int32, (n_blk, n_classes), 1)
    for b in range(n_batch):
        col = shifted[:, b:b + 1]
        out_ref[b] = jnp.where(col == iota, jnp.float32(1.0), jnp.float32(0.0))


def kernel(inputs):
    B, T = inputs.shape
    K = 256
    N = 256
    C = T // N
    P = 8
    return pl.pallas_call(
        functools.partial(_onehot_body, n_blk=N, n_classes=K, n_batch=B,
                          n_pieces=P, piece=T // P),
        grid=(C,),
        in_specs=[pl.BlockSpec((B, T), lambda j: (0, 0))],
        out_specs=pl.BlockSpec((B, N, K), lambda j: (0, j, 0)),
        out_shape=jax.ShapeDtypeStruct((B, T, K), jnp.float32),
        scratch_shapes=[pltpu.VMEM((T, B), jnp.int32)],
        compiler_params=pltpu.CompilerParams(
            dimension_semantics=("arbitrary",),
        ),
    )(inputs.astype(jnp.int32))
